# baseline (speedup 1.0000x reference)
import functools

import jax
import jax.numpy as jnp
from jax import lax
from jax.experimental import pallas as pl
from jax.experimental.pallas import tpu as pltpu
from jax.experimental.pallas import tpu_sc as plsc

NN = 10000
EE = 320000
DD = 128
EDD = 16
LL = 4
GG = 64

NW = 32
EB = 128
KB = 80
PERW = KB * EB
EPAD = NW * PERW
NACC = 10240
NBLK = 2000
EBLK = 1024


def _ln(t, g, b, eps=1e-5):
    mu = jnp.mean(t, axis=-1, keepdims=True)
    ex2 = jnp.mean(t * t, axis=-1, keepdims=True)
    r = jax.lax.rsqrt(jnp.maximum(ex2 - mu * mu, 0.0) + eps)
    return (t - mu) * r * g + b


def _gelu(t):
    return 0.5 * t * (1.0 + lax.erf(t * 0.7071067811865476))


def _embed_body(x_ref, w_ref, b_ref, g_ref, be_ref, o_ref, ob_ref):
    t = jnp.dot(x_ref[...], w_ref[...], preferred_element_type=jnp.float32)
    hn = _gelu(_ln(t + b_ref[...], g_ref[...], be_ref[...]))
    o_ref[...] = hn
    ob_ref[...] = hn.astype(jnp.bfloat16)


def _embed(x, w, b, g, be):
    return pl.pallas_call(
        _embed_body,
        grid=(NN // NBLK,),
        in_specs=[
            pl.BlockSpec((NBLK, DD), lambda i: (i, 0)),
            pl.BlockSpec((DD, DD), lambda i: (0, 0)),
            pl.BlockSpec((1, DD), lambda i: (0, 0)),
            pl.BlockSpec((1, DD), lambda i: (0, 0)),
            pl.BlockSpec((1, DD), lambda i: (0, 0)),
        ],
        out_specs=[
            pl.BlockSpec((NBLK, DD), lambda i: (i, 0)),
            pl.BlockSpec((NBLK, DD), lambda i: (i, 0)),
        ],
        out_shape=[
            jax.ShapeDtypeStruct((NN, DD), jnp.float32),
            jax.ShapeDtypeStruct((NN, DD), jnp.bfloat16),
        ],
    )(x, w, b, g, be)


def _edge_body(hm_ref, eat_ref, w1a_ref, w1b_ref, b1_ref, g1_ref,
               be1_ref, w2_ref, b2_ref, m_ref):
    hm = hm_ref[...].astype(jnp.bfloat16)
    t = jnp.dot(hm, w1a_ref[...], preferred_element_type=jnp.float32)
    t = t + lax.dot_general(eat_ref[...], w1b_ref[...],
                            (((0,), (0,)), ((), ())),
                            preferred_element_type=jnp.float32)
    t = _gelu(_ln(t + b1_ref[...], g1_ref[...], be1_ref[...]))
    m_ref[...] = jnp.dot(t.astype(jnp.bfloat16), w2_ref[...],
                         preferred_element_type=jnp.float32) + b2_ref[...]


def _edge_mlp(hm, ea_t, w1a, w1b, b1, g1, be1, w2, b2):
    ne = hm.shape[0]
    return pl.pallas_call(
        _edge_body,
        grid=(ne // EBLK,),
        in_specs=[
            pl.BlockSpec((EBLK, DD), lambda i: (i, 0)),
            pl.BlockSpec((EDD, EBLK), lambda i: (0, i)),
            pl.BlockSpec((DD, 2 * DD), lambda i: (0, 0)),
            pl.BlockSpec((EDD, 2 * DD), lambda i: (0, 0)),
            pl.BlockSpec((1, 2 * DD), lambda i: (0, 0)),
            pl.BlockSpec((1, 2 * DD), lambda i: (0, 0)),
            pl.BlockSpec((1, 2 * DD), lambda i: (0, 0)),
            pl.BlockSpec((2 * DD, DD), lambda i: (0, 0)),
            pl.BlockSpec((1, DD), lambda i: (0, 0)),
        ],
        out_specs=pl.BlockSpec((EBLK, DD), lambda i: (i, 0)),
        out_shape=jax.ShapeDtypeStruct((ne, DD), jnp.float32),
    )(hm, ea_t, w1a, w1b, b1, g1, be1, w2, b2)


def _update_body(nag, h_ref, *refs):
    (bat_ref, wh_ref, wa_ref, b_ref, g_ref, be_ref,
     o_ref, ob_ref, p_ref, c_ref) = refs[2 * nag:]
    i = pl.program_id(0)
    agg = refs[0][0]
    for a_ref in refs[1:2 * nag]:
        agg = agg + a_ref[0]
    u = jnp.dot(h_ref[...], wh_ref[...], preferred_element_type=jnp.float32)
    u = u + jnp.dot(agg, wa_ref[...], preferred_element_type=jnp.float32)
    u = _gelu(_ln(u + b_ref[...], g_ref[...], be_ref[...]))
    hn = u + h_ref[...]
    o_ref[...] = hn
    ob_ref[...] = hn.astype(jnp.bfloat16)
    oh = (lax.broadcasted_iota(jnp.int32, (GG, NBLK), 0)
          == bat_ref[0]).astype(jnp.float32)

    @pl.when(i == 0)
    def _():
        p_ref[...] = jnp.zeros_like(p_ref)
        c_ref[...] = jnp.zeros_like(c_ref)

    p_ref[...] += jnp.dot(oh, hn, preferred_element_type=jnp.float32)
    c_ref[...] += jnp.broadcast_to(
        jnp.sum(oh, axis=1, keepdims=True), (GG, DD))


def _update(h, aggs, bat3, wh, wa, b, g, be):
    nag = len(aggs)
    agg_specs = []
    agg_args = []
    for a in aggs:
        agg_specs += [pl.BlockSpec((1, NBLK, DD), lambda i: (0, i, 0)),
                      pl.BlockSpec((1, NBLK, DD), lambda i: (1, i, 0))]
        agg_args += [a, a]
    return pl.pallas_call(
        functools.partial(_update_body, nag),
        grid=(NN // NBLK,),
        in_specs=[
            pl.BlockSpec((NBLK, DD), lambda i: (i, 0)),
        ] + agg_specs + [
            pl.BlockSpec((1, 1, NBLK), lambda i: (i, 0, 0)),
            pl.BlockSpec((DD, DD), lambda i: (0, 0)),
            pl.BlockSpec((DD, DD), lambda i: (0, 0)),
            pl.BlockSpec((1, DD), lambda i: (0, 0)),
            pl.BlockSpec((1, DD), lambda i: (0, 0)),
            pl.BlockSpec((1, DD), lambda i: (0, 0)),
        ],
        out_specs=[
            pl.BlockSpec((NBLK, DD), lambda i: (i, 0)),
            pl.BlockSpec((NBLK, DD), lambda i: (i, 0)),
            pl.BlockSpec((GG, DD), lambda i: (0, 0)),
            pl.BlockSpec((GG, DD), lambda i: (0, 0)),
        ],
        out_shape=[
            jax.ShapeDtypeStruct((NN, DD), jnp.float32),
            jax.ShapeDtypeStruct((NN, DD), jnp.bfloat16),
            jax.ShapeDtypeStruct((GG, DD), jnp.float32),
            jax.ShapeDtypeStruct((GG, DD), jnp.float32),
        ],
    )(h, *agg_args, bat3, wh, wa, b, g, be)


def _final_body(p_ref, c_ref, w_ref, b_ref, g_ref, be_ref, o_ref):
    cnt = jnp.maximum(c_ref[...], 1.0)
    acc = b_ref[...]
    for l in range(LL):
        acc = acc + jnp.dot(p_ref[l] / cnt, w_ref[l],
                            preferred_element_type=jnp.float32)
    o_ref[...] = _gelu(_ln(acc, g_ref[...], be_ref[...]))


def _final(pooled, cnt, w4, b, g, be):
    return pl.pallas_call(
        _final_body,
        out_shape=jax.ShapeDtypeStruct((GG, DD), jnp.float32),
    )(pooled, cnt, w4, b, g, be)


def _sc_gather(h, idx, kb):
    mesh = plsc.VectorSubcoreMesh(core_axis_name="c", subcore_axis_name="s")
    perw = kb * EB

    @functools.partial(
        pl.kernel,
        out_type=jax.ShapeDtypeStruct((NW * perw, DD), jnp.float32),
        mesh=mesh,
        scratch_types=[
            pltpu.VMEM((perw,), jnp.int32),
            pltpu.VMEM((EB, DD), jnp.float32),
            pltpu.VMEM((EB, DD), jnp.float32),
            pltpu.VMEM((EB, DD), jnp.float32),
            pltpu.VMEM((EB, DD), jnp.float32),
            pltpu.SemaphoreType.DMA,
            pltpu.SemaphoreType.DMA,
            pltpu.SemaphoreType.DMA,
            pltpu.SemaphoreType.DMA,
            pltpu.SemaphoreType.DMA,
            pltpu.SemaphoreType.DMA,
            pltpu.SemaphoreType.DMA,
            pltpu.SemaphoreType.DMA,
        ],
    )
    def k(h_hbm, idx_hbm, out_hbm, idx_v, b0, b1, b2, b3,
          g0, g1, g2, g3, s0, s1, s2, s3):
        wid = lax.axis_index("s") * 2 + lax.axis_index("c")
        base = wid * perw
        pltpu.sync_copy(idx_hbm.at[pl.ds(base, perw)], idx_v)
        bufs = (b0, b1, b2, b3)
        gsems = (g0, g1, g2, g3)
        wsems = (s0, s1, s2, s3)

        def body(i, _):
            ks = [4 * i + j for j in range(4)]
            ds = [pltpu.async_copy(
                h_hbm.at[idx_v.at[pl.ds(k * EB, EB)]], bufs[j], gsems[j])
                for j, k in enumerate(ks)]
            ws = []
            for j, k in enumerate(ks):
                ds[j].wait()
                ws.append(pltpu.async_copy(
                    bufs[j], out_hbm.at[pl.ds(base + k * EB, EB), :],
                    wsems[j]))
            for w in ws:
                w.wait()
            return 0

        lax.fori_loop(0, kb // 4, body, 0)

    return k(h, idx)


def _sc_scatter(m, col2d, zeros_acc, kb):
    mesh = plsc.VectorSubcoreMesh(core_axis_name="c", subcore_axis_name="s")

    @functools.partial(
        pl.kernel,
        out_type=jax.ShapeDtypeStruct((2, NN, DD), jnp.float32),
        mesh=mesh,
        scratch_types=[
            pltpu.VMEM_SHARED((NACC, DD), jnp.float32),
            pltpu.VMEM((kb, EB), jnp.int32),
            pltpu.VMEM((EB, DD), jnp.float32),
            pltpu.VMEM((EB, DD), jnp.float32),
            pltpu.SemaphoreType.DMA,
            pltpu.SemaphoreType.DMA,
        ],
    )
    def k(m_hbm, col_hbm, z_hbm, out_hbm, acc_sh, idx_v, buf0, buf1, g0, g1):
        cid = lax.axis_index("c")
        sid = lax.axis_index("s")
        wid = sid * 2 + cid

        @pl.when(sid == 0)
        def _():
            pltpu.sync_copy(z_hbm, acc_sh)

        pltpu.sync_copy(col_hbm.at[wid], idx_v)
        plsc.subcore_barrier()

        def body(i, _):
            k0 = 2 * i
            k1 = k0 + 1
            e0 = (wid * kb + k0) * EB
            e1 = (wid * kb + k1) * EB
            d0 = pltpu.async_copy(m_hbm.at[pl.ds(e0, EB), :], buf0, g0)
            d1 = pltpu.async_copy(m_hbm.at[pl.ds(e1, EB), :], buf1, g1)
            d0.wait()
            pltpu.sync_copy(buf0, acc_sh.at[idx_v.at[k0]], add=True)
            d1.wait()
            pltpu.sync_copy(buf1, acc_sh.at[idx_v.at[k1]], add=True)
            return 0

        lax.fori_loop(0, kb // 2, body, 0)
        plsc.subcore_barrier()

        @pl.when(sid < 15)
        def _():
            r0 = sid * 632
            pltpu.sync_copy(acc_sh.at[pl.ds(r0, 632), :],
                            out_hbm.at[cid, pl.ds(r0, 632), :])

        @pl.when(sid == 15)
        def _():
            pltpu.sync_copy(acc_sh.at[pl.ds(9480, 520), :],
                            out_hbm.at[cid, pl.ds(9480, 520), :])

    return k(m, col2d, zeros_acc)


def _sc_fused(m, col3d, zeros_acc, h, gidx, kb):
    mesh = plsc.VectorSubcoreMesh(core_axis_name="c", subcore_axis_name="s")
    perw = kb * EB

    @functools.partial(
        pl.kernel,
        out_type=(jax.ShapeDtypeStruct((2, NN, DD), jnp.float32),
                  jax.ShapeDtypeStruct((NW * perw, DD), jnp.float32)),
        mesh=mesh,
        scratch_types=[
            pltpu.VMEM_SHARED((NN, DD), jnp.float32),
            pltpu.VMEM((kb, EB), jnp.int32),
            pltpu.VMEM((perw,), jnp.int32),
            pltpu.VMEM((EB, DD), jnp.float32),
            pltpu.VMEM((EB, DD), jnp.float32),
            pltpu.SemaphoreType.DMA,
            pltpu.SemaphoreType.DMA,
            pltpu.SemaphoreType.DMA,
        ],
    )
    def k(m_hbm, col_hbm, z_hbm, h_hbm, gidx_hbm, agg_hbm, hm_hbm,
          acc_sh, sidx_v, gidx_v, mb0, gb0, q0, g0, s0):
        cid = lax.axis_index("c")
        sid = lax.axis_index("s")
        wid = sid * 2 + cid
        base = wid * perw

        @pl.when(sid == 0)
        def _():
            pltpu.sync_copy(z_hbm.at[pl.ds(0, NN), :], acc_sh)

        pltpu.sync_copy(col_hbm.at[wid], sidx_v)
        pltpu.sync_copy(gidx_hbm.at[pl.ds(base, perw)], gidx_v)
        plsc.subcore_barrier()

        def body(k0, _):
            e0 = (wid * kb + k0) * EB
            d0 = pltpu.async_copy(
                h_hbm.at[gidx_v.at[pl.ds(k0 * EB, EB)]], gb0, g0)
            q0d = pltpu.async_copy(m_hbm.at[pl.ds(e0, EB), :], mb0, q0)
            d0.wait()
            w0 = pltpu.async_copy(
                gb0, hm_hbm.at[pl.ds(base + k0 * EB, EB), :], s0)
            q0d.wait()
            pltpu.sync_copy(mb0, acc_sh.at[sidx_v.at[k0]], add=True)
            w0.wait()
            return 0

        lax.fori_loop(0, kb, body, 0)
        plsc.subcore_barrier()

        @pl.when(sid < 15)
        def _():
            r0 = sid * 632
            pltpu.sync_copy(acc_sh.at[pl.ds(r0, 632), :],
                            agg_hbm.at[cid, pl.ds(r0, 632), :])

        @pl.when(sid == 15)
        def _():
            pltpu.sync_copy(acc_sh.at[pl.ds(9480, 520), :],
                            agg_hbm.at[cid, pl.ds(9480, 520), :])

    return k(m, col3d, zeros_acc, h, gidx)


def kernel(x, edge_index, edge_attr, batch, emb_W, emb_b, emb_g, emb_beta,
           msg_W1, msg_b1, msg_g1, msg_be1, msg_W2, msg_b2,
           upd_W, upd_b, upd_g, upd_be, comb_W, comb_b, comb_g, comb_be):
    row = edge_index[0]
    col = edge_index[1]
    npad = EPAD - EE
    row_pad = jnp.concatenate(
        [row, (jnp.arange(npad, dtype=jnp.int32) * 41) % NN])
    col_pad = jnp.concatenate(
        [col, NN + (jnp.arange(npad, dtype=jnp.int32) % (NACC - NN))])
    col2d = col_pad.reshape(NW * KB, EB)
    ea_t = jnp.concatenate(
        [edge_attr.T, jnp.zeros((EDD, npad), jnp.float32)], axis=1)
    bat3 = batch.astype(jnp.int32).reshape(NN // NBLK, 1, NBLK)
    zeros_acc = jnp.zeros((NACC, DD), jnp.float32)

    r2 = lambda v: v.reshape(1, -1)
    bf = lambda v: v.astype(jnp.bfloat16)
    ea_tb = bf(ea_t)
    h, hb = _embed(x, emb_W, r2(emb_b), r2(emb_g), r2(emb_beta))

    nch = 4
    chunk = EPAD // nch
    kbc = KB // nch
    rows = [row_pad[c * chunk:(c + 1) * chunk] for c in range(nch)]
    cols = [col2d[c * NW * kbc:(c + 1) * NW * kbc].reshape(NW, kbc, EB)
            for c in range(nch)]
    eats = [ea_tb[:, c * chunk:(c + 1) * chunk] for c in range(nch)]

    pooled = []
    cnt = None
    for l in range(LL):
        mw = (bf(msg_W1[l, :DD, :]), bf(msg_W1[l, DD:, :]),
              r2(msg_b1[l]), r2(msg_g1[l]), r2(msg_be1[l]),
              bf(msg_W2[l]), r2(msg_b2[l]))
        hms = [None] * nch
        hms[0] = _sc_gather(h, rows[0], kbc)
        hms[1] = _sc_gather(h, rows[1], kbc)
        aggs = []
        for c in range(nch):
            m = _edge_mlp(hms[c], eats[c], *mw)
            if c + 2 < nch:
                agg, hms[c + 2] = _sc_fused(m, cols[c], zeros_acc,
                                            h, rows[c + 2], kbc)
            else:
                agg = _sc_scatter(m, cols[c], zeros_acc, kbc)
            aggs.append(agg)
        h, hb, p, cnt = _update(h, aggs, bat3,
                                upd_W[l, :DD, :], upd_W[l, DD:, :],
                                r2(upd_b[l]), r2(upd_g[l]), r2(upd_be[l]))
        pooled.append(p)

    g = _final(jnp.stack(pooled), cnt, comb_W.reshape(LL, DD, DD),
               r2(comb_b), r2(comb_g), r2(comb_be))
    return (g, h)


# speedup vs baseline: 1.2092x; 1.2092x over previous
import functools

import jax
import jax.numpy as jnp
from jax import lax
from jax.experimental import pallas as pl
from jax.experimental.pallas import tpu as pltpu
from jax.experimental.pallas import tpu_sc as plsc

NN = 10000
EE = 320000
DD = 128
EDD = 16
LL = 4
GG = 64

NW = 32
EB = 128
KB = 80
PERW = KB * EB
EPAD = NW * PERW
NACC = 10240
NBLK = 2000
EBLK = 2048


def _ln(t, g, b, eps=1e-5):
    mu = jnp.mean(t, axis=-1, keepdims=True)
    ex2 = jnp.mean(t * t, axis=-1, keepdims=True)
    r = jax.lax.rsqrt(jnp.maximum(ex2 - mu * mu, 0.0) + eps)
    return (t - mu) * r * g + b


def _gelu(t):
    return 0.5 * t * (1.0 + lax.erf(t * 0.7071067811865476))


def _embed_body(x_ref, w_ref, b_ref, g_ref, be_ref, o_ref, ob_ref):
    t = jnp.dot(x_ref[...], w_ref[...], preferred_element_type=jnp.float32)
    hn = _gelu(_ln(t + b_ref[...], g_ref[...], be_ref[...]))
    o_ref[...] = hn
    ob_ref[...] = hn.astype(jnp.bfloat16)


def _embed(x, w, b, g, be):
    return pl.pallas_call(
        _embed_body,
        grid=(NN // NBLK,),
        in_specs=[
            pl.BlockSpec((NBLK, DD), lambda i: (i, 0)),
            pl.BlockSpec((DD, DD), lambda i: (0, 0)),
            pl.BlockSpec((1, DD), lambda i: (0, 0)),
            pl.BlockSpec((1, DD), lambda i: (0, 0)),
            pl.BlockSpec((1, DD), lambda i: (0, 0)),
        ],
        out_specs=[
            pl.BlockSpec((NBLK, DD), lambda i: (i, 0)),
            pl.BlockSpec((NBLK, DD), lambda i: (i, 0)),
        ],
        out_shape=[
            jax.ShapeDtypeStruct((NN, DD), jnp.float32),
            jax.ShapeDtypeStruct((NN, DD), jnp.bfloat16),
        ],
    )(x, w, b, g, be)


def _edge_body(hm_ref, eat_ref, w1a_ref, w1b_ref, b1_ref, g1_ref,
               be1_ref, w2_ref, b2_ref, m_ref):
    hm = hm_ref[...].astype(jnp.bfloat16)
    t = jnp.dot(hm, w1a_ref[...], preferred_element_type=jnp.float32)
    t = t + lax.dot_general(eat_ref[...], w1b_ref[...],
                            (((0,), (0,)), ((), ())),
                            preferred_element_type=jnp.float32)
    t = _gelu(_ln(t + b1_ref[...], g1_ref[...], be1_ref[...]))
    m_ref[...] = jnp.dot(t.astype(jnp.bfloat16), w2_ref[...],
                         preferred_element_type=jnp.float32) + b2_ref[...]


def _edge_mlp(hm, ea_t, w1a, w1b, b1, g1, be1, w2, b2):
    ne = hm.shape[0]
    return pl.pallas_call(
        _edge_body,
        grid=(ne // EBLK,),
        in_specs=[
            pl.BlockSpec((EBLK, DD), lambda i: (i, 0)),
            pl.BlockSpec((EDD, EBLK), lambda i: (0, i)),
            pl.BlockSpec((DD, 2 * DD), lambda i: (0, 0)),
            pl.BlockSpec((EDD, 2 * DD), lambda i: (0, 0)),
            pl.BlockSpec((1, 2 * DD), lambda i: (0, 0)),
            pl.BlockSpec((1, 2 * DD), lambda i: (0, 0)),
            pl.BlockSpec((1, 2 * DD), lambda i: (0, 0)),
            pl.BlockSpec((2 * DD, DD), lambda i: (0, 0)),
            pl.BlockSpec((1, DD), lambda i: (0, 0)),
        ],
        out_specs=pl.BlockSpec((EBLK, DD), lambda i: (i, 0)),
        out_shape=jax.ShapeDtypeStruct((ne, DD), jnp.float32),
    )(hm, ea_t, w1a, w1b, b1, g1, be1, w2, b2)


def _update_body(nag, h_ref, *refs):
    (bat_ref, wh_ref, wa_ref, b_ref, g_ref, be_ref,
     o_ref, ob_ref, p_ref, c_ref) = refs[2 * nag:]
    i = pl.program_id(0)
    agg = refs[0][0]
    for a_ref in refs[1:2 * nag]:
        agg = agg + a_ref[0]
    u = jnp.dot(h_ref[...], wh_ref[...], preferred_element_type=jnp.float32)
    u = u + jnp.dot(agg, wa_ref[...], preferred_element_type=jnp.float32)
    u = _gelu(_ln(u + b_ref[...], g_ref[...], be_ref[...]))
    hn = u + h_ref[...]
    o_ref[...] = hn
    ob_ref[...] = hn.astype(jnp.bfloat16)
    oh = (lax.broadcasted_iota(jnp.int32, (GG, NBLK), 0)
          == bat_ref[0]).astype(jnp.float32)

    @pl.when(i == 0)
    def _():
        p_ref[...] = jnp.zeros_like(p_ref)
        c_ref[...] = jnp.zeros_like(c_ref)

    p_ref[...] += jnp.dot(oh, hn, preferred_element_type=jnp.float32)
    c_ref[...] += jnp.broadcast_to(
        jnp.sum(oh, axis=1, keepdims=True), (GG, DD))


def _update(h, aggs, bat3, wh, wa, b, g, be):
    nag = len(aggs)
    agg_specs = []
    agg_args = []
    for a in aggs:
        agg_specs += [pl.BlockSpec((1, NBLK, DD), lambda i: (0, i, 0)),
                      pl.BlockSpec((1, NBLK, DD), lambda i: (1, i, 0))]
        agg_args += [a, a]
    return pl.pallas_call(
        functools.partial(_update_body, nag),
        grid=(NN // NBLK,),
        in_specs=[
            pl.BlockSpec((NBLK, DD), lambda i: (i, 0)),
        ] + agg_specs + [
            pl.BlockSpec((1, 1, NBLK), lambda i: (i, 0, 0)),
            pl.BlockSpec((DD, DD), lambda i: (0, 0)),
            pl.BlockSpec((DD, DD), lambda i: (0, 0)),
            pl.BlockSpec((1, DD), lambda i: (0, 0)),
            pl.BlockSpec((1, DD), lambda i: (0, 0)),
            pl.BlockSpec((1, DD), lambda i: (0, 0)),
        ],
        out_specs=[
            pl.BlockSpec((NBLK, DD), lambda i: (i, 0)),
            pl.BlockSpec((NBLK, DD), lambda i: (i, 0)),
            pl.BlockSpec((GG, DD), lambda i: (0, 0)),
            pl.BlockSpec((GG, DD), lambda i: (0, 0)),
        ],
        out_shape=[
            jax.ShapeDtypeStruct((NN, DD), jnp.float32),
            jax.ShapeDtypeStruct((NN, DD), jnp.bfloat16),
            jax.ShapeDtypeStruct((GG, DD), jnp.float32),
            jax.ShapeDtypeStruct((GG, DD), jnp.float32),
        ],
    )(h, *agg_args, bat3, wh, wa, b, g, be)


def _final_body(p_ref, c_ref, w_ref, b_ref, g_ref, be_ref, o_ref):
    cnt = jnp.maximum(c_ref[...], 1.0)
    acc = b_ref[...]
    for l in range(LL):
        acc = acc + jnp.dot(p_ref[l] / cnt, w_ref[l],
                            preferred_element_type=jnp.float32)
    o_ref[...] = _gelu(_ln(acc, g_ref[...], be_ref[...]))


def _final(pooled, cnt, w4, b, g, be):
    return pl.pallas_call(
        _final_body,
        out_shape=jax.ShapeDtypeStruct((GG, DD), jnp.float32),
    )(pooled, cnt, w4, b, g, be)


def _sc_gather(h, idx, kb):
    mesh = plsc.VectorSubcoreMesh(core_axis_name="c", subcore_axis_name="s")
    perw = kb * EB

    @functools.partial(
        pl.kernel,
        out_type=jax.ShapeDtypeStruct((NW * perw, DD), jnp.float32),
        mesh=mesh,
        scratch_types=[
            pltpu.VMEM((perw,), jnp.int32),
            pltpu.VMEM((EB, DD), jnp.float32),
            pltpu.VMEM((EB, DD), jnp.float32),
            pltpu.VMEM((EB, DD), jnp.float32),
            pltpu.VMEM((EB, DD), jnp.float32),
            pltpu.SemaphoreType.DMA,
            pltpu.SemaphoreType.DMA,
            pltpu.SemaphoreType.DMA,
            pltpu.SemaphoreType.DMA,
            pltpu.SemaphoreType.DMA,
            pltpu.SemaphoreType.DMA,
            pltpu.SemaphoreType.DMA,
            pltpu.SemaphoreType.DMA,
        ],
    )
    def k(h_hbm, idx_hbm, out_hbm, idx_v, b0, b1, b2, b3,
          g0, g1, g2, g3, s0, s1, s2, s3):
        wid = lax.axis_index("s") * 2 + lax.axis_index("c")
        base = wid * perw
        pltpu.sync_copy(idx_hbm.at[pl.ds(base, perw)], idx_v)
        bufs = (b0, b1, b2, b3)
        gsems = (g0, g1, g2, g3)
        wsems = (s0, s1, s2, s3)

        def body(i, _):
            ks = [4 * i + j for j in range(4)]
            ds = [pltpu.async_copy(
                h_hbm.at[idx_v.at[pl.ds(k * EB, EB)]], bufs[j], gsems[j])
                for j, k in enumerate(ks)]
            ws = []
            for j, k in enumerate(ks):
                ds[j].wait()
                ws.append(pltpu.async_copy(
                    bufs[j], out_hbm.at[pl.ds(base + k * EB, EB), :],
                    wsems[j]))
            for w in ws:
                w.wait()
            return 0

        lax.fori_loop(0, kb // 4, body, 0)

    return k(h, idx)


def _sc_scatter(m, col2d, zeros_acc, kb):
    mesh = plsc.VectorSubcoreMesh(core_axis_name="c", subcore_axis_name="s")

    @functools.partial(
        pl.kernel,
        out_type=jax.ShapeDtypeStruct((2, NN, DD), jnp.float32),
        mesh=mesh,
        scratch_types=[
            pltpu.VMEM_SHARED((NACC, DD), jnp.float32),
            pltpu.VMEM((kb, EB), jnp.int32),
            pltpu.VMEM((EB, DD), jnp.float32),
            pltpu.VMEM((EB, DD), jnp.float32),
            pltpu.SemaphoreType.DMA,
            pltpu.SemaphoreType.DMA,
        ],
    )
    def k(m_hbm, col_hbm, z_hbm, out_hbm, acc_sh, idx_v, buf0, buf1, g0, g1):
        cid = lax.axis_index("c")
        sid = lax.axis_index("s")
        wid = sid * 2 + cid

        @pl.when(sid == 0)
        def _():
            pltpu.sync_copy(z_hbm, acc_sh)

        pltpu.sync_copy(col_hbm.at[wid], idx_v)
        plsc.subcore_barrier()

        def body(i, _):
            k0 = 2 * i
            k1 = k0 + 1
            e0 = (wid * kb + k0) * EB
            e1 = (wid * kb + k1) * EB
            d0 = pltpu.async_copy(m_hbm.at[pl.ds(e0, EB), :], buf0, g0)
            d1 = pltpu.async_copy(m_hbm.at[pl.ds(e1, EB), :], buf1, g1)
            d0.wait()
            pltpu.sync_copy(buf0, acc_sh.at[idx_v.at[k0]], add=True)
            d1.wait()
            pltpu.sync_copy(buf1, acc_sh.at[idx_v.at[k1]], add=True)
            return 0

        lax.fori_loop(0, kb // 2, body, 0)
        plsc.subcore_barrier()

        @pl.when(sid < 15)
        def _():
            r0 = sid * 632
            pltpu.sync_copy(acc_sh.at[pl.ds(r0, 632), :],
                            out_hbm.at[cid, pl.ds(r0, 632), :])

        @pl.when(sid == 15)
        def _():
            pltpu.sync_copy(acc_sh.at[pl.ds(9480, 520), :],
                            out_hbm.at[cid, pl.ds(9480, 520), :])

    return k(m, col2d, zeros_acc)


def _sc_fused(m, col3d, zeros_acc, h, gidx, kb):
    mesh = plsc.VectorSubcoreMesh(core_axis_name="c", subcore_axis_name="s")
    perw = kb * EB

    @functools.partial(
        pl.kernel,
        out_type=(jax.ShapeDtypeStruct((2, NN, DD), jnp.float32),
                  jax.ShapeDtypeStruct((NW * perw, DD), jnp.float32)),
        mesh=mesh,
        scratch_types=[
            pltpu.VMEM_SHARED((NN, DD), jnp.float32),
            pltpu.VMEM((kb, EB), jnp.int32),
            pltpu.VMEM((perw,), jnp.int32),
            pltpu.VMEM((EB, DD), jnp.float32),
            pltpu.VMEM((EB, DD), jnp.float32),
            pltpu.SemaphoreType.DMA,
            pltpu.SemaphoreType.DMA,
            pltpu.SemaphoreType.DMA,
        ],
    )
    def k(m_hbm, col_hbm, z_hbm, h_hbm, gidx_hbm, agg_hbm, hm_hbm,
          acc_sh, sidx_v, gidx_v, mb0, gb0, q0, g0, s0):
        cid = lax.axis_index("c")
        sid = lax.axis_index("s")
        wid = sid * 2 + cid
        base = wid * perw

        @pl.when(sid == 0)
        def _():
            pltpu.sync_copy(z_hbm.at[pl.ds(0, NN), :], acc_sh)

        pltpu.sync_copy(col_hbm.at[wid], sidx_v)
        pltpu.sync_copy(gidx_hbm.at[pl.ds(base, perw)], gidx_v)
        plsc.subcore_barrier()

        def body(k0, _):
            e0 = (wid * kb + k0) * EB
            d0 = pltpu.async_copy(
                h_hbm.at[gidx_v.at[pl.ds(k0 * EB, EB)]], gb0, g0)
            q0d = pltpu.async_copy(m_hbm.at[pl.ds(e0, EB), :], mb0, q0)
            d0.wait()
            w0 = pltpu.async_copy(
                gb0, hm_hbm.at[pl.ds(base + k0 * EB, EB), :], s0)
            q0d.wait()
            pltpu.sync_copy(mb0, acc_sh.at[sidx_v.at[k0]], add=True)
            w0.wait()
            return 0

        lax.fori_loop(0, kb, body, 0)
        plsc.subcore_barrier()

        @pl.when(sid < 15)
        def _():
            r0 = sid * 632
            pltpu.sync_copy(acc_sh.at[pl.ds(r0, 632), :],
                            agg_hbm.at[cid, pl.ds(r0, 632), :])

        @pl.when(sid == 15)
        def _():
            pltpu.sync_copy(acc_sh.at[pl.ds(9480, 520), :],
                            agg_hbm.at[cid, pl.ds(9480, 520), :])

    return k(m, col3d, zeros_acc, h, gidx)


def kernel(x, edge_index, edge_attr, batch, emb_W, emb_b, emb_g, emb_beta,
           msg_W1, msg_b1, msg_g1, msg_be1, msg_W2, msg_b2,
           upd_W, upd_b, upd_g, upd_be, comb_W, comb_b, comb_g, comb_be):
    row = edge_index[0]
    col = edge_index[1]
    npad = EPAD - EE
    row_pad = jnp.concatenate(
        [row, (jnp.arange(npad, dtype=jnp.int32) * 41) % NN])
    col_pad = jnp.concatenate(
        [col, NN + (jnp.arange(npad, dtype=jnp.int32) % (NACC - NN))])
    col2d = col_pad.reshape(NW * KB, EB)
    ea_t = jnp.concatenate(
        [edge_attr.T, jnp.zeros((EDD, npad), jnp.float32)], axis=1)
    bat3 = batch.astype(jnp.int32).reshape(NN // NBLK, 1, NBLK)
    zeros_acc = jnp.zeros((NACC, DD), jnp.float32)

    r2 = lambda v: v.reshape(1, -1)
    bf = lambda v: v.astype(jnp.bfloat16)
    ea_tb = bf(ea_t)
    h, hb = _embed(x, emb_W, r2(emb_b), r2(emb_g), r2(emb_beta))

    nch = 4
    chunk = EPAD // nch
    kbc = KB // nch
    rows = [row_pad[c * chunk:(c + 1) * chunk] for c in range(nch)]
    cols = [col2d[c * NW * kbc:(c + 1) * NW * kbc].reshape(NW, kbc, EB)
            for c in range(nch)]
    eats = [ea_tb[:, c * chunk:(c + 1) * chunk] for c in range(nch)]

    pooled = []
    cnt = None
    for l in range(LL):
        mw = (bf(msg_W1[l, :DD, :]), bf(msg_W1[l, DD:, :]),
              r2(msg_b1[l]), r2(msg_g1[l]), r2(msg_be1[l]),
              bf(msg_W2[l]), r2(msg_b2[l]))
        hms = [None] * nch
        hms[0] = _sc_gather(h, rows[0], kbc)
        hms[1] = _sc_gather(h, rows[1], kbc)
        aggs = []
        for c in range(nch):
            m = _edge_mlp(hms[c], eats[c], *mw)
            if c + 2 < nch:
                agg, hms[c + 2] = _sc_fused(m, cols[c], zeros_acc,
                                            h, rows[c + 2], kbc)
            else:
                agg = _sc_scatter(m, cols[c], zeros_acc, kbc)
            aggs.append(agg)
        h, hb, p, cnt = _update(h, aggs, bat3,
                                upd_W[l, :DD, :], upd_W[l, DD:, :],
                                r2(upd_b[l]), r2(upd_g[l]), r2(upd_be[l]))
        pooled.append(p)

    g = _final(jnp.stack(pooled), cnt, comb_W.reshape(LL, DD, DD),
               r2(comb_b), r2(comb_g), r2(comb_be))
    return (g, h)


# speedup vs baseline: 1.2612x; 1.0430x over previous
import functools

import jax
import jax.numpy as jnp
from jax import lax
from jax.experimental import pallas as pl
from jax.experimental.pallas import tpu as pltpu
from jax.experimental.pallas import tpu_sc as plsc

NN = 10000
EE = 320000
DD = 128
EDD = 16
LL = 4
GG = 64

NW = 32
EB = 128
KB = 80
PERW = KB * EB
EPAD = NW * PERW
NACC = 10240
NBLK = 2000
EBLK = 4096


def _ln(t, g, b, eps=1e-5):
    mu = jnp.mean(t, axis=-1, keepdims=True)
    ex2 = jnp.mean(t * t, axis=-1, keepdims=True)
    r = jax.lax.rsqrt(jnp.maximum(ex2 - mu * mu, 0.0) + eps)
    return (t - mu) * r * g + b


def _gelu(t):
    return 0.5 * t * (1.0 + lax.erf(t * 0.7071067811865476))


def _embed_body(x_ref, w_ref, b_ref, g_ref, be_ref, o_ref, ob_ref):
    t = jnp.dot(x_ref[...], w_ref[...], preferred_element_type=jnp.float32)
    hn = _gelu(_ln(t + b_ref[...], g_ref[...], be_ref[...]))
    o_ref[...] = hn
    ob_ref[...] = hn.astype(jnp.bfloat16)


def _embed(x, w, b, g, be):
    return pl.pallas_call(
        _embed_body,
        grid=(NN // NBLK,),
        in_specs=[
            pl.BlockSpec((NBLK, DD), lambda i: (i, 0)),
            pl.BlockSpec((DD, DD), lambda i: (0, 0)),
            pl.BlockSpec((1, DD), lambda i: (0, 0)),
            pl.BlockSpec((1, DD), lambda i: (0, 0)),
            pl.BlockSpec((1, DD), lambda i: (0, 0)),
        ],
        out_specs=[
            pl.BlockSpec((NBLK, DD), lambda i: (i, 0)),
            pl.BlockSpec((NBLK, DD), lambda i: (i, 0)),
        ],
        out_shape=[
            jax.ShapeDtypeStruct((NN, DD), jnp.float32),
            jax.ShapeDtypeStruct((NN, DD), jnp.bfloat16),
        ],
    )(x, w, b, g, be)


def _edge_body(hm_ref, eat_ref, w1a_ref, w1b_ref, b1_ref, g1_ref,
               be1_ref, w2_ref, b2_ref, m_ref):
    hm = hm_ref[...].astype(jnp.bfloat16)
    t = jnp.dot(hm, w1a_ref[...], preferred_element_type=jnp.float32)
    t = t + lax.dot_general(eat_ref[...], w1b_ref[...],
                            (((0,), (0,)), ((), ())),
                            preferred_element_type=jnp.float32)
    t = _gelu(_ln(t + b1_ref[...], g1_ref[...], be1_ref[...]))
    m_ref[...] = jnp.dot(t.astype(jnp.bfloat16), w2_ref[...],
                         preferred_element_type=jnp.float32) + b2_ref[...]


def _edge_mlp(hm, ea_t, w1a, w1b, b1, g1, be1, w2, b2):
    ne = hm.shape[0]
    return pl.pallas_call(
        _edge_body,
        grid=(ne // EBLK,),
        in_specs=[
            pl.BlockSpec((EBLK, DD), lambda i: (i, 0)),
            pl.BlockSpec((EDD, EBLK), lambda i: (0, i)),
            pl.BlockSpec((DD, 2 * DD), lambda i: (0, 0)),
            pl.BlockSpec((EDD, 2 * DD), lambda i: (0, 0)),
            pl.BlockSpec((1, 2 * DD), lambda i: (0, 0)),
            pl.BlockSpec((1, 2 * DD), lambda i: (0, 0)),
            pl.BlockSpec((1, 2 * DD), lambda i: (0, 0)),
            pl.BlockSpec((2 * DD, DD), lambda i: (0, 0)),
            pl.BlockSpec((1, DD), lambda i: (0, 0)),
        ],
        out_specs=pl.BlockSpec((EBLK, DD), lambda i: (i, 0)),
        out_shape=jax.ShapeDtypeStruct((ne, DD), jnp.float32),
    )(hm, ea_t, w1a, w1b, b1, g1, be1, w2, b2)


def _update_body(nag, h_ref, *refs):
    (bat_ref, wh_ref, wa_ref, b_ref, g_ref, be_ref,
     o_ref, ob_ref, p_ref, c_ref) = refs[2 * nag:]
    i = pl.program_id(0)
    agg = refs[0][0]
    for a_ref in refs[1:2 * nag]:
        agg = agg + a_ref[0]
    u = jnp.dot(h_ref[...], wh_ref[...], preferred_element_type=jnp.float32)
    u = u + jnp.dot(agg, wa_ref[...], preferred_element_type=jnp.float32)
    u = _gelu(_ln(u + b_ref[...], g_ref[...], be_ref[...]))
    hn = u + h_ref[...]
    o_ref[...] = hn
    ob_ref[...] = hn.astype(jnp.bfloat16)
    oh = (lax.broadcasted_iota(jnp.int32, (GG, NBLK), 0)
          == bat_ref[0]).astype(jnp.float32)

    @pl.when(i == 0)
    def _():
        p_ref[...] = jnp.zeros_like(p_ref)
        c_ref[...] = jnp.zeros_like(c_ref)

    p_ref[...] += jnp.dot(oh, hn, preferred_element_type=jnp.float32)
    c_ref[...] += jnp.broadcast_to(
        jnp.sum(oh, axis=1, keepdims=True), (GG, DD))


def _update(h, aggs, bat3, wh, wa, b, g, be):
    nag = len(aggs)
    agg_specs = []
    agg_args = []
    for a in aggs:
        agg_specs += [pl.BlockSpec((1, NBLK, DD), lambda i: (0, i, 0)),
                      pl.BlockSpec((1, NBLK, DD), lambda i: (1, i, 0))]
        agg_args += [a, a]
    return pl.pallas_call(
        functools.partial(_update_body, nag),
        grid=(NN // NBLK,),
        in_specs=[
            pl.BlockSpec((NBLK, DD), lambda i: (i, 0)),
        ] + agg_specs + [
            pl.BlockSpec((1, 1, NBLK), lambda i: (i, 0, 0)),
            pl.BlockSpec((DD, DD), lambda i: (0, 0)),
            pl.BlockSpec((DD, DD), lambda i: (0, 0)),
            pl.BlockSpec((1, DD), lambda i: (0, 0)),
            pl.BlockSpec((1, DD), lambda i: (0, 0)),
            pl.BlockSpec((1, DD), lambda i: (0, 0)),
        ],
        out_specs=[
            pl.BlockSpec((NBLK, DD), lambda i: (i, 0)),
            pl.BlockSpec((NBLK, DD), lambda i: (i, 0)),
            pl.BlockSpec((GG, DD), lambda i: (0, 0)),
            pl.BlockSpec((GG, DD), lambda i: (0, 0)),
        ],
        out_shape=[
            jax.ShapeDtypeStruct((NN, DD), jnp.float32),
            jax.ShapeDtypeStruct((NN, DD), jnp.bfloat16),
            jax.ShapeDtypeStruct((GG, DD), jnp.float32),
            jax.ShapeDtypeStruct((GG, DD), jnp.float32),
        ],
    )(h, *agg_args, bat3, wh, wa, b, g, be)


def _final_body(p_ref, c_ref, w_ref, b_ref, g_ref, be_ref, o_ref):
    cnt = jnp.maximum(c_ref[...], 1.0)
    acc = b_ref[...]
    for l in range(LL):
        acc = acc + jnp.dot(p_ref[l] / cnt, w_ref[l],
                            preferred_element_type=jnp.float32)
    o_ref[...] = _gelu(_ln(acc, g_ref[...], be_ref[...]))


def _final(pooled, cnt, w4, b, g, be):
    return pl.pallas_call(
        _final_body,
        out_shape=jax.ShapeDtypeStruct((GG, DD), jnp.float32),
    )(pooled, cnt, w4, b, g, be)


def _sc_gather(h, idx, kb):
    mesh = plsc.VectorSubcoreMesh(core_axis_name="c", subcore_axis_name="s")
    perw = kb * EB

    @functools.partial(
        pl.kernel,
        out_type=jax.ShapeDtypeStruct((NW * perw, DD), jnp.float32),
        mesh=mesh,
        scratch_types=[
            pltpu.VMEM((perw,), jnp.int32),
            pltpu.VMEM((EB, DD), jnp.float32),
            pltpu.VMEM((EB, DD), jnp.float32),
            pltpu.VMEM((EB, DD), jnp.float32),
            pltpu.VMEM((EB, DD), jnp.float32),
            pltpu.SemaphoreType.DMA,
            pltpu.SemaphoreType.DMA,
            pltpu.SemaphoreType.DMA,
            pltpu.SemaphoreType.DMA,
            pltpu.SemaphoreType.DMA,
            pltpu.SemaphoreType.DMA,
            pltpu.SemaphoreType.DMA,
            pltpu.SemaphoreType.DMA,
        ],
    )
    def k(h_hbm, idx_hbm, out_hbm, idx_v, b0, b1, b2, b3,
          g0, g1, g2, g3, s0, s1, s2, s3):
        wid = lax.axis_index("s") * 2 + lax.axis_index("c")
        base = wid * perw
        pltpu.sync_copy(idx_hbm.at[pl.ds(base, perw)], idx_v)
        bufs = (b0, b1, b2, b3)
        gsems = (g0, g1, g2, g3)
        wsems = (s0, s1, s2, s3)

        def body(i, _):
            ks = [4 * i + j for j in range(4)]
            ds = [pltpu.async_copy(
                h_hbm.at[idx_v.at[pl.ds(k * EB, EB)]], bufs[j], gsems[j])
                for j, k in enumerate(ks)]
            ws = []
            for j, k in enumerate(ks):
                ds[j].wait()
                ws.append(pltpu.async_copy(
                    bufs[j], out_hbm.at[pl.ds(base + k * EB, EB), :],
                    wsems[j]))
            for w in ws:
                w.wait()
            return 0

        lax.fori_loop(0, kb // 4, body, 0)

    return k(h, idx)


def _sc_scatter(m, col2d, zeros_acc, kb):
    mesh = plsc.VectorSubcoreMesh(core_axis_name="c", subcore_axis_name="s")

    @functools.partial(
        pl.kernel,
        out_type=jax.ShapeDtypeStruct((2, NN, DD), jnp.float32),
        mesh=mesh,
        scratch_types=[
            pltpu.VMEM_SHARED((NACC, DD), jnp.float32),
            pltpu.VMEM((kb, EB), jnp.int32),
            pltpu.VMEM((EB, DD), jnp.float32),
            pltpu.VMEM((EB, DD), jnp.float32),
            pltpu.SemaphoreType.DMA,
            pltpu.SemaphoreType.DMA,
        ],
    )
    def k(m_hbm, col_hbm, z_hbm, out_hbm, acc_sh, idx_v, buf0, buf1, g0, g1):
        cid = lax.axis_index("c")
        sid = lax.axis_index("s")
        wid = sid * 2 + cid

        @pl.when(sid == 0)
        def _():
            pltpu.sync_copy(z_hbm, acc_sh)

        pltpu.sync_copy(col_hbm.at[wid], idx_v)
        plsc.subcore_barrier()

        def body(i, _):
            k0 = 2 * i
            k1 = k0 + 1
            e0 = (wid * kb + k0) * EB
            e1 = (wid * kb + k1) * EB
            d0 = pltpu.async_copy(m_hbm.at[pl.ds(e0, EB), :], buf0, g0)
            d1 = pltpu.async_copy(m_hbm.at[pl.ds(e1, EB), :], buf1, g1)
            d0.wait()
            pltpu.sync_copy(buf0, acc_sh.at[idx_v.at[k0]], add=True)
            d1.wait()
            pltpu.sync_copy(buf1, acc_sh.at[idx_v.at[k1]], add=True)
            return 0

        lax.fori_loop(0, kb // 2, body, 0)
        plsc.subcore_barrier()

        @pl.when(sid < 15)
        def _():
            r0 = sid * 632
            pltpu.sync_copy(acc_sh.at[pl.ds(r0, 632), :],
                            out_hbm.at[cid, pl.ds(r0, 632), :])

        @pl.when(sid == 15)
        def _():
            pltpu.sync_copy(acc_sh.at[pl.ds(9480, 520), :],
                            out_hbm.at[cid, pl.ds(9480, 520), :])

    return k(m, col2d, zeros_acc)


def _sc_fused(m, col3d, zeros_acc, h, gidx, kb):
    mesh = plsc.VectorSubcoreMesh(core_axis_name="c", subcore_axis_name="s")
    perw = kb * EB

    @functools.partial(
        pl.kernel,
        out_type=(jax.ShapeDtypeStruct((2, NN, DD), jnp.float32),
                  jax.ShapeDtypeStruct((NW * perw, DD), jnp.float32)),
        mesh=mesh,
        scratch_types=[
            pltpu.VMEM_SHARED((NN, DD), jnp.float32),
            pltpu.VMEM((kb, EB), jnp.int32),
            pltpu.VMEM((perw,), jnp.int32),
            pltpu.VMEM((EB, DD), jnp.float32),
            pltpu.VMEM((EB, DD), jnp.float32),
            pltpu.SemaphoreType.DMA,
            pltpu.SemaphoreType.DMA,
            pltpu.SemaphoreType.DMA,
        ],
    )
    def k(m_hbm, col_hbm, z_hbm, h_hbm, gidx_hbm, agg_hbm, hm_hbm,
          acc_sh, sidx_v, gidx_v, mb0, gb0, q0, g0, s0):
        cid = lax.axis_index("c")
        sid = lax.axis_index("s")
        wid = sid * 2 + cid
        base = wid * perw

        @pl.when(sid == 0)
        def _():
            pltpu.sync_copy(z_hbm.at[pl.ds(0, NN), :], acc_sh)

        pltpu.sync_copy(col_hbm.at[wid], sidx_v)
        pltpu.sync_copy(gidx_hbm.at[pl.ds(base, perw)], gidx_v)
        plsc.subcore_barrier()

        def body(k0, _):
            e0 = (wid * kb + k0) * EB
            d0 = pltpu.async_copy(
                h_hbm.at[gidx_v.at[pl.ds(k0 * EB, EB)]], gb0, g0)
            q0d = pltpu.async_copy(m_hbm.at[pl.ds(e0, EB), :], mb0, q0)
            d0.wait()
            w0 = pltpu.async_copy(
                gb0, hm_hbm.at[pl.ds(base + k0 * EB, EB), :], s0)
            q0d.wait()
            pltpu.sync_copy(mb0, acc_sh.at[sidx_v.at[k0]], add=True)
            w0.wait()
            return 0

        lax.fori_loop(0, kb, body, 0)
        plsc.subcore_barrier()

        @pl.when(sid < 15)
        def _():
            r0 = sid * 632
            pltpu.sync_copy(acc_sh.at[pl.ds(r0, 632), :],
                            agg_hbm.at[cid, pl.ds(r0, 632), :])

        @pl.when(sid == 15)
        def _():
            pltpu.sync_copy(acc_sh.at[pl.ds(9480, 520), :],
                            agg_hbm.at[cid, pl.ds(9480, 520), :])

    return k(m, col3d, zeros_acc, h, gidx)


def kernel(x, edge_index, edge_attr, batch, emb_W, emb_b, emb_g, emb_beta,
           msg_W1, msg_b1, msg_g1, msg_be1, msg_W2, msg_b2,
           upd_W, upd_b, upd_g, upd_be, comb_W, comb_b, comb_g, comb_be):
    row = edge_index[0]
    col = edge_index[1]
    npad = EPAD - EE
    row_pad = jnp.concatenate(
        [row, (jnp.arange(npad, dtype=jnp.int32) * 41) % NN])
    col_pad = jnp.concatenate(
        [col, NN + (jnp.arange(npad, dtype=jnp.int32) % (NACC - NN))])
    col2d = col_pad.reshape(NW * KB, EB)
    ea_t = jnp.concatenate(
        [edge_attr.T, jnp.zeros((EDD, npad), jnp.float32)], axis=1)
    bat3 = batch.astype(jnp.int32).reshape(NN // NBLK, 1, NBLK)
    zeros_acc = jnp.zeros((NACC, DD), jnp.float32)

    r2 = lambda v: v.reshape(1, -1)
    bf = lambda v: v.astype(jnp.bfloat16)
    ea_tb = bf(ea_t)
    h, hb = _embed(x, emb_W, r2(emb_b), r2(emb_g), r2(emb_beta))

    nch = 4
    chunk = EPAD // nch
    kbc = KB // nch
    rows = [row_pad[c * chunk:(c + 1) * chunk] for c in range(nch)]
    cols = [col2d[c * NW * kbc:(c + 1) * NW * kbc].reshape(NW, kbc, EB)
            for c in range(nch)]
    eats = [ea_tb[:, c * chunk:(c + 1) * chunk] for c in range(nch)]

    pooled = []
    cnt = None
    for l in range(LL):
        mw = (bf(msg_W1[l, :DD, :]), bf(msg_W1[l, DD:, :]),
              r2(msg_b1[l]), r2(msg_g1[l]), r2(msg_be1[l]),
              bf(msg_W2[l]), r2(msg_b2[l]))
        hms = [None] * nch
        hms[0] = _sc_gather(h, rows[0], kbc)
        hms[1] = _sc_gather(h, rows[1], kbc)
        aggs = []
        for c in range(nch):
            m = _edge_mlp(hms[c], eats[c], *mw)
            if c + 2 < nch:
                agg, hms[c + 2] = _sc_fused(m, cols[c], zeros_acc,
                                            h, rows[c + 2], kbc)
            else:
                agg = _sc_scatter(m, cols[c], zeros_acc, kbc)
            aggs.append(agg)
        h, hb, p, cnt = _update(h, aggs, bat3,
                                upd_W[l, :DD, :], upd_W[l, DD:, :],
                                r2(upd_b[l]), r2(upd_g[l]), r2(upd_be[l]))
        pooled.append(p)

    g = _final(jnp.stack(pooled), cnt, comb_W.reshape(LL, DD, DD),
               r2(comb_b), r2(comb_g), r2(comb_be))
    return (g, h)
